# baseline (speedup 1.0000x reference)
import functools

import jax
import jax.numpy as jnp
from jax.experimental import pallas as pl
from jax.experimental.pallas import tpu as pltpu


def _pos_body(row_ref, cola_ref, colb_ref, depth_ref, out_ref, *, hb, wh):
    di = pl.program_id(0)
    cola = cola_ref[...]
    colb = colb_ref[...]
    zu = jnp.zeros((wh, 64), jnp.float32)
    u = jnp.concatenate([cola, zu, zu, colb, zu, zu], axis=1)
    row = row_ref[...]
    depth = jnp.broadcast_to(depth_ref[pl.ds(di, 1), :], (hb, 64))
    zv = jnp.zeros((hb, 64), jnp.float32)
    v = jnp.concatenate([zv, row, depth, zv, row, depth], axis=1)
    out_ref[...] = (u[None, :, :] + v[:, None, :])[None]


def kernel(scan, row_weight, col_weight, depth_weight):
    d, em, h, w = scan.shape
    hb = 32
    n_h = h // hb
    wh = w // 2
    col_a = col_weight[0:w:2]
    col_b = col_weight[1:w:2]
    body = functools.partial(_pos_body, hb=hb, wh=wh)
    out = pl.pallas_call(
        body,
        grid=(d, n_h),
        in_specs=[
            pl.BlockSpec((hb, 64), lambda di, hi: (hi, 0)),
            pl.BlockSpec((wh, 64), lambda di, hi: (0, 0)),
            pl.BlockSpec((wh, 64), lambda di, hi: (0, 0)),
            pl.BlockSpec((40, 64), lambda di, hi: (0, 0)),
        ],
        out_specs=pl.BlockSpec((1, hb, wh, 384), lambda di, hi: (di, hi, 0, 0)),
        out_shape=jax.ShapeDtypeStruct((d, h, wh, 384), jnp.float32),
        compiler_params=pltpu.CompilerParams(
            dimension_semantics=("parallel", "parallel")),
    )(row_weight, col_a, col_b, depth_weight)
    return out.reshape(d, h, w, 192)


# speedup vs baseline: 1.0093x; 1.0093x over previous
import functools

import jax
import jax.numpy as jnp
from jax.experimental import pallas as pl
from jax.experimental.pallas import tpu as pltpu


def _pos_body(row_ref, uflat_ref, depth_ref, out_ref, *, hb, wh):
    di = pl.program_id(0)
    row = row_ref[...]
    depth = jnp.broadcast_to(depth_ref[pl.ds(di, 1), :], (hb, 64))
    zv = jnp.zeros((hb, 64), jnp.float32)
    v = jnp.concatenate([zv, row, depth, zv, row, depth], axis=1)
    vt = jnp.tile(v, (1, wh))
    u = jnp.broadcast_to(uflat_ref[...], (hb, wh * 384))
    out_ref[...] = (vt + u)[None]


def kernel(scan, row_weight, col_weight, depth_weight):
    d, em, h, w = scan.shape
    hb = 32
    n_h = h // hb
    wh = w // 2
    lanes = wh * 384
    zu = jnp.zeros((wh, 64), jnp.float32)
    u_flat = jnp.concatenate(
        [col_weight[0:w:2], zu, zu, col_weight[1:w:2], zu, zu], axis=1
    ).reshape(1, lanes)
    body = functools.partial(_pos_body, hb=hb, wh=wh)
    out = pl.pallas_call(
        body,
        grid=(d, n_h),
        in_specs=[
            pl.BlockSpec((hb, 64), lambda di, hi: (hi, 0)),
            pl.BlockSpec((1, lanes), lambda di, hi: (0, 0)),
            pl.BlockSpec((40, 64), lambda di, hi: (0, 0)),
        ],
        out_specs=pl.BlockSpec((1, hb, lanes), lambda di, hi: (di, hi, 0)),
        out_shape=jax.ShapeDtypeStruct((d, h, lanes), jnp.float32),
        compiler_params=pltpu.CompilerParams(
            dimension_semantics=("parallel", "parallel")),
    )(row_weight, u_flat, depth_weight)
    return out.reshape(d, h, w, 192)


# speedup vs baseline: 3.6697x; 3.6361x over previous
import functools

import jax
import jax.numpy as jnp
from jax.experimental import pallas as pl
from jax.experimental.pallas import tpu as pltpu


def _pos_body(row_ref, col_ref, depth_ref, out_ref, scratch_ref, sem, *, hb, w):
    di = pl.program_id(0)
    hi = pl.program_id(1)
    col = col_ref[0:w, :]
    row = row_ref[...]
    depth = depth_ref[pl.ds(di, 1), :]
    zc = jnp.zeros((w, 64), jnp.float32)
    zr = jnp.zeros((hb, 64), jnp.float32)
    u = jnp.concatenate(
        [col, zc, jnp.broadcast_to(depth, (w, 64))], axis=-1)
    v = jnp.concatenate([zr, row, zr], axis=-1)
    scratch_ref[...] = u[None, :, :] + v[:, None, :]
    copy = pltpu.make_async_copy(
        scratch_ref, out_ref.at[di, pl.ds(hi * hb, hb)], sem)
    copy.start()
    copy.wait()


def kernel(scan, row_weight, col_weight, depth_weight):
    d, em, h, w = scan.shape
    hb = 32
    n_h = h // hb
    body = functools.partial(_pos_body, hb=hb, w=w)
    out = pl.pallas_call(
        body,
        grid=(d, n_h),
        in_specs=[
            pl.BlockSpec((hb, 64), lambda di, hi: (hi, 0)),
            pl.BlockSpec((256, 64), lambda di, hi: (0, 0)),
            pl.BlockSpec((40, 64), lambda di, hi: (0, 0)),
        ],
        out_specs=pl.BlockSpec(memory_space=pltpu.MemorySpace.HBM),
        out_shape=jax.ShapeDtypeStruct((d, h, w, 192), jnp.float32),
        scratch_shapes=[
            pltpu.VMEM((hb, w, 192), jnp.float32),
            pltpu.SemaphoreType.DMA,
        ],
        compiler_params=pltpu.CompilerParams(
            dimension_semantics=("parallel", "parallel")),
    )(row_weight, col_weight, depth_weight)
    return out


# speedup vs baseline: 4.2029x; 1.1453x over previous
import functools

import jax
import jax.numpy as jnp
from jax.experimental import pallas as pl
from jax.experimental.pallas import tpu as pltpu

_NSLOT = 4


def _copy_for_step(step, out_ref, scratch_ref, sems, *, hb, n_h):
    di = step // n_h
    hi = step % n_h
    slot = step % _NSLOT
    return pltpu.make_async_copy(
        scratch_ref.at[slot],
        out_ref.at[di, pl.ds(hi * hb, hb)],
        sems.at[slot],
    )


def _pos_body(row_ref, col_ref, depth_ref, out_ref, scratch_ref, sems,
              *, hb, w, n_h, total):
    di = pl.program_id(0)
    hi = pl.program_id(1)
    step = di * n_h + hi
    slot = step % _NSLOT

    @pl.when(step >= _NSLOT)
    def _():
        _copy_for_step(step - _NSLOT, out_ref, scratch_ref, sems,
                       hb=hb, n_h=n_h).wait()

    col = col_ref[0:w, :]
    row = row_ref[...]
    depth = depth_ref[pl.ds(di, 1), :]
    zc = jnp.zeros((w, 64), jnp.float32)
    zr = jnp.zeros((hb, 64), jnp.float32)
    u = jnp.concatenate(
        [col, zc, jnp.broadcast_to(depth, (w, 64))], axis=-1)
    v = jnp.concatenate([zr, row, zr], axis=-1)
    scratch_ref[slot] = u[None, :, :] + v[:, None, :]

    _copy_for_step(step, out_ref, scratch_ref, sems, hb=hb, n_h=n_h).start()

    @pl.when(step == total - 1)
    def _():
        for j in range(_NSLOT):
            _copy_for_step(total - _NSLOT + j, out_ref, scratch_ref, sems,
                           hb=hb, n_h=n_h).wait()


def kernel(scan, row_weight, col_weight, depth_weight):
    d, em, h, w = scan.shape
    hb = 32
    n_h = h // hb
    total = d * n_h
    body = functools.partial(_pos_body, hb=hb, w=w, n_h=n_h, total=total)
    out = pl.pallas_call(
        body,
        grid=(d, n_h),
        in_specs=[
            pl.BlockSpec((hb, 64), lambda di, hi: (hi, 0)),
            pl.BlockSpec((256, 64), lambda di, hi: (0, 0)),
            pl.BlockSpec((40, 64), lambda di, hi: (0, 0)),
        ],
        out_specs=pl.BlockSpec(memory_space=pltpu.MemorySpace.HBM),
        out_shape=jax.ShapeDtypeStruct((d, h, w, 192), jnp.float32),
        scratch_shapes=[
            pltpu.VMEM((_NSLOT, hb, w, 192), jnp.float32),
            pltpu.SemaphoreType.DMA((_NSLOT,)),
        ],
        compiler_params=pltpu.CompilerParams(
            dimension_semantics=("arbitrary", "arbitrary")),
    )(row_weight, col_weight, depth_weight)
    return out


# speedup vs baseline: 4.2162x; 1.0032x over previous
import functools

import jax
import jax.numpy as jnp
from jax.experimental import pallas as pl
from jax.experimental.pallas import tpu as pltpu

_NSLOT = 4


def _copies_for_step(step, out_ref, scratch_ref, sems, *, hb, n_h):
    di = step // n_h
    hi = step % n_h
    slot = step % _NSLOT
    head = pltpu.make_async_copy(
        scratch_ref.at[slot, :, :, 0:128],
        out_ref.at[di, pl.ds(hi * hb, hb), :, 0:128],
        sems.at[slot, 0],
    )
    tail = pltpu.make_async_copy(
        scratch_ref.at[slot, :, :, 128:192],
        out_ref.at[di, pl.ds(hi * hb, hb), :, 128:192],
        sems.at[slot, 1],
    )
    return head, tail


def _pos_body(row_ref, col_ref, depth_ref, out_ref, scratch_ref, sems,
              *, hb, w, n_h, total):
    di = pl.program_id(0)
    hi = pl.program_id(1)
    step = di * n_h + hi
    slot = step % _NSLOT

    @pl.when(step >= _NSLOT)
    def _():
        ch, ct = _copies_for_step(step - _NSLOT, out_ref, scratch_ref, sems,
                                  hb=hb, n_h=n_h)
        ch.wait()
        ct.wait()

    col = col_ref[0:w, :]
    row = row_ref[...]
    depth = depth_ref[pl.ds(di, 1), :]
    zc = jnp.zeros((w, 64), jnp.float32)
    zr = jnp.zeros((hb, 64), jnp.float32)
    u = jnp.concatenate(
        [col, zc, jnp.broadcast_to(depth, (w, 64))], axis=-1)
    v = jnp.concatenate([zr, row, zr], axis=-1)
    scratch_ref[slot] = u[None, :, :] + v[:, None, :]

    ch, ct = _copies_for_step(step, out_ref, scratch_ref, sems, hb=hb, n_h=n_h)
    ch.start()
    ct.start()

    @pl.when(step == total - 1)
    def _():
        for j in range(_NSLOT):
            dh, dt = _copies_for_step(total - _NSLOT + j, out_ref, scratch_ref,
                                      sems, hb=hb, n_h=n_h)
            dh.wait()
            dt.wait()


def kernel(scan, row_weight, col_weight, depth_weight):
    d, em, h, w = scan.shape
    hb = 32
    n_h = h // hb
    total = d * n_h
    body = functools.partial(_pos_body, hb=hb, w=w, n_h=n_h, total=total)
    out = pl.pallas_call(
        body,
        grid=(d, n_h),
        in_specs=[
            pl.BlockSpec((hb, 64), lambda di, hi: (hi, 0)),
            pl.BlockSpec((256, 64), lambda di, hi: (0, 0)),
            pl.BlockSpec((40, 64), lambda di, hi: (0, 0)),
        ],
        out_specs=pl.BlockSpec(memory_space=pltpu.MemorySpace.HBM),
        out_shape=jax.ShapeDtypeStruct((d, h, w, 192), jnp.float32),
        scratch_shapes=[
            pltpu.VMEM((_NSLOT, hb, w, 192), jnp.float32),
            pltpu.SemaphoreType.DMA((_NSLOT, 2)),
        ],
        compiler_params=pltpu.CompilerParams(
            dimension_semantics=("arbitrary", "arbitrary")),
    )(row_weight, col_weight, depth_weight)
    return out
